# Initial kernel scaffold; baseline (speedup 1.0000x reference)
import jax
import jax.numpy as jnp
from jax.experimental import pallas as pl


def kernel(x, item_emb_matrix, positional_emb):
    raise NotImplementedError("write your pallas kernel here")


# speedup vs baseline: 1.1790x; 1.1790x over previous
import functools

import jax
import jax.numpy as jnp
from jax import lax
from jax.experimental import pallas as pl
from jax.experimental.pallas import tpu as pltpu
from jax.experimental.pallas import tpu_sc as plsc

BATCH = 4096
SEQ = 200
DIM = 32
ROWS = BATCH * SEQ


def _build(num_workers, rows_per_w, nchunk):
    mesh = plsc.VectorSubcoreMesh(core_axis_name="c", subcore_axis_name="s")

    @functools.partial(
        pl.kernel,
        mesh=mesh,
        out_type=jax.ShapeDtypeStruct((ROWS, DIM), jnp.float32),
        compiler_params=pltpu.CompilerParams(use_tc_tiling_on_sc=False),
        scratch_types=[
            pltpu.VMEM((SEQ,), jnp.int32),
            pltpu.VMEM((SEQ, DIM), jnp.float32),
            pltpu.VMEM((SEQ, DIM), jnp.float32),
            pltpu.SemaphoreType.DMA,
        ],
    )
    def k(table_hbm, idx_hbm, pos_hbm, out_hbm, idx_v, rows_v, pos_v, sem):
        nc = 2
        wid = lax.axis_index("s") * nc + lax.axis_index("c")
        base = wid * rows_per_w
        pltpu.sync_copy(pos_hbm, pos_v)

        def chunk_body(c, carry):
            off = base + c * SEQ
            pltpu.sync_copy(idx_hbm.at[pl.ds(off, SEQ)], idx_v)
            pltpu.async_copy(table_hbm.at[idx_v], rows_v, sem).wait()

            def row_body(r, carry2):
                rows_v[r, pl.ds(0, 16)] = (
                    rows_v[r, pl.ds(0, 16)] + pos_v[r, pl.ds(0, 16)]
                )
                rows_v[r, pl.ds(16, 16)] = (
                    rows_v[r, pl.ds(16, 16)] + pos_v[r, pl.ds(16, 16)]
                )
                return carry2

            lax.fori_loop(0, SEQ, row_body, 0)
            pltpu.sync_copy(rows_v, out_hbm.at[pl.ds(off, SEQ)])
            return carry

        lax.fori_loop(0, nchunk, chunk_body, 0)

    return k


def kernel(x, item_emb_matrix, positional_emb):
    idx = x.reshape(ROWS).astype(jnp.int32)
    info = plsc.get_sparse_core_info()
    num_workers = info.num_cores * info.num_subcores
    rows_per_w = ROWS // num_workers
    nchunk = rows_per_w // SEQ
    out = _build(num_workers, rows_per_w, nchunk)(
        item_emb_matrix, idx, positional_emb
    )
    return out.reshape(BATCH, SEQ, DIM)


# speedup vs baseline: 1.4917x; 1.2652x over previous
import functools

import jax
import jax.numpy as jnp
from jax import lax
from jax.experimental import pallas as pl
from jax.experimental.pallas import tpu as pltpu
from jax.experimental.pallas import tpu_sc as plsc

BATCH = 4096
SEQ = 200
DIM = 32
ROWS = BATCH * SEQ
SEQ_PER_CHUNK = 4
CHUNK = SEQ * SEQ_PER_CHUNK
NBUF = 2


def _build(num_workers, rows_per_w, nchunk):
    mesh = plsc.VectorSubcoreMesh(core_axis_name="c", subcore_axis_name="s")

    @functools.partial(
        pl.kernel,
        mesh=mesh,
        out_type=jax.ShapeDtypeStruct((ROWS, DIM), jnp.float32),
        compiler_params=pltpu.CompilerParams(use_tc_tiling_on_sc=False),
        scratch_types=[
            pltpu.VMEM((rows_per_w,), jnp.int32),
            pltpu.VMEM((NBUF, CHUNK, DIM), jnp.float32),
            pltpu.VMEM((SEQ, DIM), jnp.float32),
            pltpu.SemaphoreType.DMA,
            pltpu.SemaphoreType.DMA,
            pltpu.SemaphoreType.DMA,
            pltpu.SemaphoreType.DMA,
        ],
    )
    def k(table_hbm, idx_hbm, pos_hbm, out_hbm, idx_v, rows_v, pos_v,
          gsem0, gsem1, ssem0, ssem1):
        nc = 2
        wid = lax.axis_index("s") * nc + lax.axis_index("c")
        base = wid * rows_per_w
        pltpu.sync_copy(idx_hbm.at[pl.ds(base, rows_per_w)], idx_v)
        pltpu.sync_copy(pos_hbm, pos_v)
        gsems = (gsem0, gsem1)
        ssems = (ssem0, ssem1)

        def gather(c):
            b = c % NBUF
            return pltpu.make_async_copy(
                table_hbm.at[idx_v.at[pl.ds(c * CHUNK, CHUNK)]],
                rows_v.at[b],
                gsems[b],
            )

        def store(c):
            b = c % NBUF
            return pltpu.make_async_copy(
                rows_v.at[b],
                out_hbm.at[pl.ds(base + c * CHUNK, CHUNK)],
                ssems[b],
            )

        gather(0).start()

        for c in range(nchunk):
            b = c % NBUF
            gather(c).wait()
            if c + 1 < nchunk:
                if c + 1 >= NBUF:
                    store(c + 1 - NBUF).wait()
                gather(c + 1).start()

            def seq_body(l, carry):
                p0 = pos_v[l, pl.ds(0, 16)]
                p1 = pos_v[l, pl.ds(16, 16)]
                for s in range(SEQ_PER_CHUNK):
                    r = s * SEQ + l
                    rows_v[b, r, pl.ds(0, 16)] = (
                        rows_v[b, r, pl.ds(0, 16)] + p0
                    )
                    rows_v[b, r, pl.ds(16, 16)] = (
                        rows_v[b, r, pl.ds(16, 16)] + p1
                    )
                return carry

            lax.fori_loop(0, SEQ, seq_body, 0)
            store(c).start()

        for c in range(max(0, nchunk - NBUF), nchunk):
            store(c).wait()

    return k


def kernel(x, item_emb_matrix, positional_emb):
    idx = x.reshape(ROWS).astype(jnp.int32)
    info = plsc.get_sparse_core_info()
    num_workers = info.num_cores * info.num_subcores
    rows_per_w = ROWS // num_workers
    nchunk = rows_per_w // CHUNK
    out = _build(num_workers, rows_per_w, nchunk)(
        item_emb_matrix, idx, positional_emb
    )
    return out.reshape(BATCH, SEQ, DIM)
